# baseline (speedup 1.0000x reference)
import jax
import jax.numpy as jnp
from jax.experimental import pallas as pl
from jax.experimental.pallas import tpu as pltpu

NUM_EXPERTS = 64
EMBED_DIM = 2048
NUM_TOKENS = 16384

BT = 1024
NBUF = 4


def _router_body(x_hbm, wt_hbm, onehot_ref, pmax_ref, logits_ref,
                 xbuf, wt_vmem, sems, wt_sem):
    i = pl.program_id(0)
    nblk = pl.num_programs(0)

    def start_copy(blk):
        slot = jax.lax.rem(blk, NBUF)
        pltpu.make_async_copy(
            x_hbm.at[pl.ds(blk * BT, BT), :],
            xbuf.at[slot],
            sems.at[slot],
        ).start()

    @pl.when(i == 0)
    def _():
        pltpu.make_async_copy(wt_hbm, wt_vmem, wt_sem).start()
        for b in range(NBUF - 1):
            start_copy(b)
        pltpu.make_async_copy(wt_hbm, wt_vmem, wt_sem).wait()

    @pl.when(i + NBUF - 1 < nblk)
    def _():
        start_copy(i + NBUF - 1)

    slot = jax.lax.rem(i, NBUF)
    pltpu.make_async_copy(
        x_hbm.at[pl.ds(i * BT, BT), :],
        xbuf.at[slot],
        sems.at[slot],
    ).wait()

    x = xbuf[slot]
    wt = wt_vmem[...]
    logits = jax.lax.dot_general(
        x, wt, (((1,), (1,)), ((), ())),
        preferred_element_type=jnp.float32)
    rows = pl.ds(i * BT, BT)
    logits_ref[rows, :] = logits
    m = jnp.max(logits, axis=1, keepdims=True)
    s = jnp.sum(jnp.exp(logits - m), axis=1, keepdims=True)
    pmax_ref[rows, :] = 1.0 / s
    idx = jnp.argmax(logits, axis=1)
    iota = jax.lax.broadcasted_iota(jnp.int32, logits.shape, 1)
    onehot_ref[rows, :] = (iota == idx[:, None]).astype(jnp.int32)


@jax.jit
def kernel(hidden_states, W):
    grid = (NUM_TOKENS // BT,)
    onehot, pmax, logits = pl.pallas_call(
        _router_body,
        grid=grid,
        in_specs=[
            pl.BlockSpec(memory_space=pl.ANY),
            pl.BlockSpec(memory_space=pl.ANY),
        ],
        out_specs=[
            pl.BlockSpec((NUM_TOKENS, NUM_EXPERTS), lambda i: (0, 0)),
            pl.BlockSpec((NUM_TOKENS, 1), lambda i: (0, 0)),
            pl.BlockSpec((NUM_TOKENS, NUM_EXPERTS), lambda i: (0, 0)),
        ],
        out_shape=[
            jax.ShapeDtypeStruct((NUM_TOKENS, NUM_EXPERTS), jnp.int32),
            jax.ShapeDtypeStruct((NUM_TOKENS, 1), jnp.float32),
            jax.ShapeDtypeStruct((NUM_TOKENS, NUM_EXPERTS), jnp.float32),
        ],
        scratch_shapes=[
            pltpu.VMEM((NBUF, BT, EMBED_DIM), jnp.float32),
            pltpu.VMEM((NUM_EXPERTS, EMBED_DIM), jnp.float32),
            pltpu.SemaphoreType.DMA((NBUF,)),
            pltpu.SemaphoreType.DMA,
        ],
    )(hidden_states, W)
    return (onehot, pmax, logits)


# speedup vs baseline: 1.0066x; 1.0066x over previous
import jax
import jax.numpy as jnp
from jax.experimental import pallas as pl
from jax.experimental.pallas import tpu as pltpu

NUM_EXPERTS = 64
EMBED_DIM = 2048
NUM_TOKENS = 16384

HB = 512
BIG = 2 * HB
NBUF = 4


def _router_body(x_hbm, w_hbm, onehot_ref, pmax_ref, logits_ref,
                 xbuf, w_vmem, sems, w_sem):
    i = pl.program_id(0)
    nblk = pl.num_programs(0)

    def start_copy(blk):
        slot = jax.lax.rem(blk, NBUF)
        pltpu.make_async_copy(
            x_hbm.at[pl.ds(blk * BIG, BIG), :],
            xbuf.at[slot],
            sems.at[slot],
        ).start()

    @pl.when(i == 0)
    def _():
        pltpu.make_async_copy(w_hbm, w_vmem, w_sem).start()
        for b in range(NBUF - 1):
            start_copy(b)
        pltpu.make_async_copy(w_hbm, w_vmem, w_sem).wait()

    @pl.when(i + NBUF - 1 < nblk)
    def _():
        start_copy(i + NBUF - 1)

    slot = jax.lax.rem(i, NBUF)
    pltpu.make_async_copy(
        x_hbm.at[pl.ds(i * BIG, BIG), :],
        xbuf.at[slot],
        sems.at[slot],
    ).wait()

    w = w_vmem[...]
    la = jax.lax.dot_general(
        xbuf[slot, :HB, :], w, (((1,), (1,)), ((), ())),
        preferred_element_type=jnp.float32)
    lb = jax.lax.dot_general(
        xbuf[slot, HB:, :], w, (((1,), (1,)), ((), ())),
        preferred_element_type=jnp.float32)

    def tail(larr, rb):
        rows = pl.ds(rb, HB)
        logits_ref[rows, :] = larr
        m = jnp.max(larr, axis=1, keepdims=True)
        s = jnp.sum(jnp.exp(larr - m), axis=1, keepdims=True)
        pmax_ref[rows, :] = 1.0 / s
        idx = jnp.argmax(larr, axis=1)
        iota = jax.lax.broadcasted_iota(jnp.int32, larr.shape, 1)
        onehot_ref[rows, :] = (iota == idx[:, None]).astype(jnp.int32)

    tail(la, i * BIG)
    tail(lb, i * BIG + HB)


@jax.jit
def kernel(hidden_states, W):
    grid = (NUM_TOKENS // BIG,)
    onehot, pmax, logits = pl.pallas_call(
        _router_body,
        grid=grid,
        in_specs=[
            pl.BlockSpec(memory_space=pl.ANY),
            pl.BlockSpec(memory_space=pl.ANY),
        ],
        out_specs=[
            pl.BlockSpec((NUM_TOKENS, NUM_EXPERTS), lambda i: (0, 0)),
            pl.BlockSpec((NUM_TOKENS, 1), lambda i: (0, 0)),
            pl.BlockSpec((NUM_TOKENS, NUM_EXPERTS), lambda i: (0, 0)),
        ],
        out_shape=[
            jax.ShapeDtypeStruct((NUM_TOKENS, NUM_EXPERTS), jnp.int32),
            jax.ShapeDtypeStruct((NUM_TOKENS, 1), jnp.float32),
            jax.ShapeDtypeStruct((NUM_TOKENS, NUM_EXPERTS), jnp.float32),
        ],
        scratch_shapes=[
            pltpu.VMEM((NBUF, BIG, EMBED_DIM), jnp.float32),
            pltpu.VMEM((NUM_EXPERTS, EMBED_DIM), jnp.float32),
            pltpu.SemaphoreType.DMA((NBUF,)),
            pltpu.SemaphoreType.DMA,
        ],
    )(hidden_states, W)
    return (onehot, pmax, logits)


# speedup vs baseline: 1.0180x; 1.0113x over previous
import jax
import jax.numpy as jnp
from jax.experimental import pallas as pl
from jax.experimental.pallas import tpu as pltpu

NUM_EXPERTS = 64
EMBED_DIM = 2048
NUM_TOKENS = 16384

BT = 512
NBUF = 8


def _router_body(x_hbm, wt_hbm, onehot_ref, pmax_ref, logits_ref,
                 xbuf, wt_vmem, sems, wt_sem):
    i = pl.program_id(0)
    nblk = pl.num_programs(0)

    def start_copy(blk):
        slot = jax.lax.rem(blk, NBUF)
        pltpu.make_async_copy(
            x_hbm.at[pl.ds(blk * BT, BT), :],
            xbuf.at[slot],
            sems.at[slot],
        ).start()

    @pl.when(i == 0)
    def _():
        pltpu.make_async_copy(wt_hbm, wt_vmem, wt_sem).start()
        for b in range(NBUF - 1):
            start_copy(b)
        pltpu.make_async_copy(wt_hbm, wt_vmem, wt_sem).wait()

    @pl.when(i + NBUF - 1 < nblk)
    def _():
        start_copy(i + NBUF - 1)

    slot = jax.lax.rem(i, NBUF)
    pltpu.make_async_copy(
        x_hbm.at[pl.ds(i * BT, BT), :],
        xbuf.at[slot],
        sems.at[slot],
    ).wait()

    x = xbuf[slot]
    wt = wt_vmem[...]
    logits = jax.lax.dot_general(
        x, wt, (((1,), (1,)), ((), ())),
        preferred_element_type=jnp.float32)
    rows = pl.ds(i * BT, BT)
    logits_ref[rows, :] = logits
    m = jnp.max(logits, axis=1, keepdims=True)
    s = jnp.sum(jnp.exp(logits - m), axis=1, keepdims=True)
    pmax_ref[rows, :] = 1.0 / s
    idx = jnp.argmax(logits, axis=1)
    iota = jax.lax.broadcasted_iota(jnp.int32, logits.shape, 1)
    onehot_ref[rows, :] = (iota == idx[:, None]).astype(jnp.int32)


@jax.jit
def kernel(hidden_states, W):
    grid = (NUM_TOKENS // BT,)
    onehot, pmax, logits = pl.pallas_call(
        _router_body,
        grid=grid,
        in_specs=[
            pl.BlockSpec(memory_space=pl.ANY),
            pl.BlockSpec(memory_space=pl.ANY),
        ],
        out_specs=[
            pl.BlockSpec((NUM_TOKENS, NUM_EXPERTS), lambda i: (0, 0)),
            pl.BlockSpec((NUM_TOKENS, 1), lambda i: (0, 0)),
            pl.BlockSpec((NUM_TOKENS, NUM_EXPERTS), lambda i: (0, 0)),
        ],
        out_shape=[
            jax.ShapeDtypeStruct((NUM_TOKENS, NUM_EXPERTS), jnp.int32),
            jax.ShapeDtypeStruct((NUM_TOKENS, 1), jnp.float32),
            jax.ShapeDtypeStruct((NUM_TOKENS, NUM_EXPERTS), jnp.float32),
        ],
        scratch_shapes=[
            pltpu.VMEM((NBUF, BT, EMBED_DIM), jnp.float32),
            pltpu.VMEM((NUM_EXPERTS, EMBED_DIM), jnp.float32),
            pltpu.SemaphoreType.DMA((NBUF,)),
            pltpu.SemaphoreType.DMA,
        ],
    )(hidden_states, W)
    return (onehot, pmax, logits)
